# Initial kernel scaffold; baseline (speedup 1.0000x reference)
import jax
import jax.numpy as jnp
from jax.experimental import pallas as pl


def kernel(x, edge_index, lin_w, lin_bias, fc_w, fc_b):
    raise NotImplementedError("write your pallas kernel here")


# speedup vs baseline: 15.9958x; 15.9958x over previous
import functools

import jax
import jax.numpy as jnp
from jax import lax
from jax.experimental import pallas as pl
from jax.experimental.pallas import tpu as pltpu
from jax.experimental.pallas import tpu_sc as plsc

N = 10000
E = 320000
C = 128
N_P = 10240

NC = 2
NS = 16
NW = NC * NS
EPW = E // NW
CHUNK = 80
NCHUNK = EPW // CHUNK
RCHUNK = N_P // CHUNK
RPS = RCHUNK // NS

_mesh = plsc.VectorSubcoreMesh(
    core_axis_name="c", subcore_axis_name="s", num_cores=NC, num_subcores=NS
)


def _fill_vec16(ref, nwords, value):
    val = jnp.full((16,), value, dtype=jnp.float32)

    @pl.loop(0, nwords // 16)
    def _(i):
        ref[pl.ds(i * 16, 16)] = val


@functools.partial(
    pl.kernel,
    out_type=jax.ShapeDtypeStruct((NC * N_P,), jnp.float32),
    mesh=_mesh,
    scratch_types=[
        pltpu.VMEM((CHUNK,), jnp.int32),
        pltpu.VMEM((CHUNK,), jnp.float32),
        pltpu.VMEM((CHUNK,), jnp.float32),
        pltpu.VMEM_SHARED((N_P,), jnp.float32),
    ],
)
def _deg_kernel(col_hbm, out_hbm, cidx, ones_v, tmp_v, deg_sh):
    cid = lax.axis_index("c")
    sid = lax.axis_index("s")
    wid = sid * NC + cid

    _fill_vec16(ones_v, CHUNK, 1.0)
    _fill_vec16(tmp_v, CHUNK, 0.0)

    @pl.loop(sid * RPS, (sid + 1) * RPS)
    def _(j):
        pltpu.sync_copy(tmp_v, deg_sh.at[pl.ds(j * CHUNK, CHUNK)])

    plsc.subcore_barrier()

    base = wid * EPW

    @pl.loop(0, NCHUNK)
    def _(j):
        off = pl.multiple_of(base + j * CHUNK, 8)
        pltpu.sync_copy(col_hbm.at[pl.ds(off, CHUNK)], cidx)
        pltpu.sync_copy(ones_v, deg_sh.at[cidx], add=True)

    plsc.subcore_barrier()

    @pl.loop(sid * RPS, (sid + 1) * RPS)
    def _(j):
        o = pl.multiple_of(j * CHUNK, 8)
        pltpu.sync_copy(deg_sh.at[pl.ds(o, CHUNK)], tmp_v)
        oo = pl.multiple_of(cid * N_P + o, 8)
        pltpu.sync_copy(tmp_v, out_hbm.at[pl.ds(oo, CHUNK)])


@functools.partial(
    pl.kernel,
    out_type=jax.ShapeDtypeStruct((NC, N_P, C), jnp.float32),
    mesh=_mesh,
    scratch_types=[
        pltpu.VMEM((CHUNK,), jnp.int32),
        pltpu.VMEM((CHUNK,), jnp.int32),
        pltpu.VMEM((CHUNK, C), jnp.float32),
        pltpu.VMEM_SHARED((N_P, C), jnp.float32),
        pltpu.SemaphoreType.DMA,
    ],
)
def _agg_kernel(g_hbm, row_hbm, col_hbm, out_hbm, ridx, cidx, rows_v, acc_sh, sem):
    cid = lax.axis_index("c")
    sid = lax.axis_index("s")
    wid = sid * NC + cid

    zval = jnp.zeros((16,), jnp.float32)

    @pl.loop(0, CHUNK)
    def _(r):
        for c16 in range(C // 16):
            rows_v[r, pl.ds(c16 * 16, 16)] = zval

    @pl.loop(sid * RPS, (sid + 1) * RPS)
    def _(j):
        pltpu.sync_copy(rows_v, acc_sh.at[pl.ds(j * CHUNK, CHUNK), :])

    plsc.subcore_barrier()

    base = wid * EPW

    @pl.loop(0, NCHUNK)
    def _(j):
        off = pl.multiple_of(base + j * CHUNK, 8)
        pltpu.sync_copy(row_hbm.at[pl.ds(off, CHUNK)], ridx)
        pltpu.sync_copy(col_hbm.at[pl.ds(off, CHUNK)], cidx)
        pltpu.async_copy(g_hbm.at[ridx], rows_v, sem).wait()
        pltpu.sync_copy(rows_v, acc_sh.at[cidx], add=True)

    plsc.subcore_barrier()

    @pl.loop(sid * RPS, (sid + 1) * RPS)
    def _(j):
        o = pl.multiple_of(j * CHUNK, 8)
        pltpu.sync_copy(acc_sh.at[pl.ds(o, CHUNK), :], rows_v)
        pltpu.sync_copy(rows_v, out_hbm.at[cid, pl.ds(o, CHUNK), :])


B = 1280
GRID = N_P // B


def _k2_body(deg_ref, x_ref, w_ref, g_ref):
    dp = deg_ref[...]
    s = jnp.sqrt(dp[0] + dp[1])
    h = lax.dot_general(
        x_ref[...], w_ref[...], (((1,), (1,)), ((), ())),
        preferred_element_type=jnp.float32,
    )
    g_ref[...] = h * s[:, None]


def _k4_body(s_part_ref, deg_ref, lb_ref, fw_ref, fb_ref, out_ref):
    sp = s_part_ref[...]
    st = sp[0] + sp[1]
    dp = deg_ref[...]
    s = jnp.sqrt(dp[0] + dp[1])
    a = st * s[:, None] + lb_ref[...][None, :]
    out_ref[...] = (
        lax.dot_general(
            a, fw_ref[...], (((1,), (1,)), ((), ())),
            preferred_element_type=jnp.float32,
        )
        + fb_ref[...][None, :]
    )


_k2 = pl.pallas_call(
    _k2_body,
    out_shape=jax.ShapeDtypeStruct((N_P, C), jnp.float32),
    grid=(GRID,),
    in_specs=[
        pl.BlockSpec((NC, B), lambda i: (0, i)),
        pl.BlockSpec((B, C), lambda i: (i, 0)),
        pl.BlockSpec((C, C), lambda i: (0, 0)),
    ],
    out_specs=pl.BlockSpec((B, C), lambda i: (i, 0)),
)

_k4 = pl.pallas_call(
    _k4_body,
    out_shape=jax.ShapeDtypeStruct((N_P, C), jnp.float32),
    grid=(GRID,),
    in_specs=[
        pl.BlockSpec((NC, B, C), lambda i: (0, i, 0)),
        pl.BlockSpec((NC, B), lambda i: (0, i)),
        pl.BlockSpec((C,), lambda i: (0,)),
        pl.BlockSpec((C, C), lambda i: (0, 0)),
        pl.BlockSpec((C,), lambda i: (0,)),
    ],
    out_specs=pl.BlockSpec((B, C), lambda i: (i, 0)),
)


def kernel(x, edge_index, lin_w, lin_bias, fc_w, fc_b):
    row = edge_index[0]
    col = edge_index[1]
    x_p = jnp.pad(x, ((0, N_P - N), (0, 0)))

    deg_part = _deg_kernel(col).reshape(NC, N_P)
    g = _k2(deg_part, x_p, lin_w)
    s_part = _agg_kernel(g, row, col)
    out = _k4(s_part, deg_part, lin_bias, fc_w, fc_b)
    return out[:N]


# speedup vs baseline: 38.2655x; 2.3922x over previous
import functools

import jax
import jax.numpy as jnp
from jax import lax
from jax.experimental import pallas as pl
from jax.experimental.pallas import tpu as pltpu
from jax.experimental.pallas import tpu_sc as plsc

N = 10000
E = 320000
C = 128
N_P = 10240

NC = 2
NS = 16
NW = NC * NS
EPW = E // NW
CHUNK = 80
NCHUNK = EPW // CHUNK
RCHUNK = N_P // CHUNK
RPS = RCHUNK // NS

_mesh = plsc.VectorSubcoreMesh(
    core_axis_name="c", subcore_axis_name="s", num_cores=NC, num_subcores=NS
)


def _fill_vec16(ref, nwords, value):
    val = jnp.full((16,), value, dtype=jnp.float32)

    @pl.loop(0, nwords // 16)
    def _(i):
        ref[pl.ds(i * 16, 16)] = val


NPS = N_P // NS


@functools.partial(
    pl.kernel,
    out_type=jax.ShapeDtypeStruct((NC * N_P,), jnp.float32),
    mesh=_mesh,
    scratch_types=[
        pltpu.VMEM((NCHUNK, CHUNK), jnp.int32),
        pltpu.VMEM((CHUNK,), jnp.float32),
        pltpu.VMEM((NPS,), jnp.float32),
        pltpu.VMEM_SHARED((N_P,), jnp.float32),
        pltpu.SemaphoreType.DMA,
        pltpu.SemaphoreType.DMA,
    ],
)
def _deg_kernel(col3d_hbm, out_hbm, cidx_all, ones_v, tmp_v, deg_sh, semi, sems):
    cid = lax.axis_index("c")
    sid = lax.axis_index("s")
    wid = sid * NC + cid

    idx_load = pltpu.async_copy(col3d_hbm.at[wid], cidx_all, semi)

    _fill_vec16(ones_v, CHUNK, 1.0)
    _fill_vec16(tmp_v, NPS, 0.0)

    pltpu.sync_copy(tmp_v, deg_sh.at[pl.ds(sid * NPS, NPS)])
    idx_load.wait()
    plsc.subcore_barrier()

    @pl.loop(0, NCHUNK // 5)
    def _(m):
        ds_ = []
        for k in range(5):
            ds_.append(
                pltpu.async_copy(
                    ones_v, deg_sh.at[cidx_all.at[m * 5 + k]], sems, add=True
                )
            )
        for d in ds_:
            d.wait()

    plsc.subcore_barrier()

    pltpu.sync_copy(deg_sh.at[pl.ds(sid * NPS, NPS)], tmp_v)
    oo = pl.multiple_of(cid * N_P + sid * NPS, 8)
    pltpu.sync_copy(tmp_v, out_hbm.at[pl.ds(oo, NPS)])


@functools.partial(
    pl.kernel,
    out_type=jax.ShapeDtypeStruct((NC, N_P, C), jnp.float32),
    mesh=_mesh,
    scratch_types=[
        pltpu.VMEM((EPW,), jnp.int32),
        pltpu.VMEM((NCHUNK, CHUNK), jnp.int32),
        pltpu.VMEM((CHUNK, C), jnp.float32),
        pltpu.VMEM((CHUNK, C), jnp.float32),
        pltpu.VMEM_SHARED((N_P, C), jnp.float32),
        pltpu.SemaphoreType.DMA,
        pltpu.SemaphoreType.DMA,
        pltpu.SemaphoreType.DMA,
    ],
)
def _agg_kernel(
    g_hbm, row_hbm, col3d_hbm, out_hbm,
    ridx_all, cidx_all, rows_a, rows_b, acc_sh, sem_a, sem_b, semi,
):
    cid = lax.axis_index("c")
    sid = lax.axis_index("s")
    wid = sid * NC + cid

    roff = pl.multiple_of(wid * EPW, 8)
    rload = pltpu.async_copy(row_hbm.at[pl.ds(roff, EPW)], ridx_all, semi)
    cload = pltpu.async_copy(col3d_hbm.at[wid], cidx_all, semi)

    zval = jnp.zeros((16,), jnp.float32)

    @pl.loop(0, CHUNK)
    def _(r):
        for c16 in range(C // 16):
            rows_a[r, pl.ds(c16 * 16, 16)] = zval

    @pl.loop(sid * RPS, (sid + 1) * RPS)
    def _(j):
        pltpu.sync_copy(rows_a, acc_sh.at[pl.ds(j * CHUNK, CHUNK), :])

    rload.wait()
    cload.wait()
    plsc.subcore_barrier()

    pltpu.async_copy(g_hbm.at[ridx_all.at[pl.ds(0, CHUNK)]], rows_a, sem_a)
    pltpu.async_copy(g_hbm.at[ridx_all.at[pl.ds(CHUNK, CHUNK)]], rows_b, sem_b)

    @pl.loop(0, (NCHUNK - 1) // 2)
    def _(p):
        j = 2 * p
        pltpu.make_async_copy(g_hbm.at[ridx_all.at[pl.ds(j * CHUNK, CHUNK)]], rows_a, sem_a).wait()
        pltpu.sync_copy(rows_a, acc_sh.at[cidx_all.at[j]], add=True)
        pltpu.async_copy(g_hbm.at[ridx_all.at[pl.ds((j + 2) * CHUNK, CHUNK)]], rows_a, sem_a)

        pltpu.make_async_copy(g_hbm.at[ridx_all.at[pl.ds((j + 1) * CHUNK, CHUNK)]], rows_b, sem_b).wait()
        pltpu.sync_copy(rows_b, acc_sh.at[cidx_all.at[j + 1]], add=True)

        @pl.when(j + 3 < NCHUNK)
        def _():
            pltpu.async_copy(g_hbm.at[ridx_all.at[pl.ds((j + 3) * CHUNK, CHUNK)]], rows_b, sem_b)

    last = NCHUNK - 1
    pltpu.make_async_copy(g_hbm.at[ridx_all.at[pl.ds(last * CHUNK, CHUNK)]], rows_a, sem_a).wait()
    pltpu.sync_copy(rows_a, acc_sh.at[cidx_all.at[last]], add=True)

    plsc.subcore_barrier()

    @pl.loop(sid * RPS, (sid + 1) * RPS)
    def _(j):
        o = pl.multiple_of(j * CHUNK, 8)
        pltpu.sync_copy(acc_sh.at[pl.ds(o, CHUNK), :], rows_a)
        pltpu.sync_copy(rows_a, out_hbm.at[cid, pl.ds(o, CHUNK), :])


B = 1280
GRID = N_P // B


def _k2_body(deg_ref, x_ref, w_ref, g_ref):
    dp = deg_ref[...]
    s = jnp.sqrt(dp[0] + dp[1])
    h = lax.dot_general(
        x_ref[...], w_ref[...], (((1,), (1,)), ((), ())),
        preferred_element_type=jnp.float32,
    )
    g_ref[...] = h * s[:, None]


def _k4_body(s_part_ref, deg_ref, lb_ref, fw_ref, fb_ref, out_ref):
    sp = s_part_ref[...]
    st = sp[0] + sp[1]
    dp = deg_ref[...]
    s = jnp.sqrt(dp[0] + dp[1])
    a = st * s[:, None] + lb_ref[...][None, :]
    out_ref[...] = (
        lax.dot_general(
            a, fw_ref[...], (((1,), (1,)), ((), ())),
            preferred_element_type=jnp.float32,
        )
        + fb_ref[...][None, :]
    )


_k2 = pl.pallas_call(
    _k2_body,
    out_shape=jax.ShapeDtypeStruct((N_P, C), jnp.float32),
    grid=(GRID,),
    in_specs=[
        pl.BlockSpec((NC, B), lambda i: (0, i)),
        pl.BlockSpec((B, C), lambda i: (i, 0)),
        pl.BlockSpec((C, C), lambda i: (0, 0)),
    ],
    out_specs=pl.BlockSpec((B, C), lambda i: (i, 0)),
)

_k4 = pl.pallas_call(
    _k4_body,
    out_shape=jax.ShapeDtypeStruct((N_P, C), jnp.float32),
    grid=(GRID,),
    in_specs=[
        pl.BlockSpec((NC, B, C), lambda i: (0, i, 0)),
        pl.BlockSpec((NC, B), lambda i: (0, i)),
        pl.BlockSpec((C,), lambda i: (0,)),
        pl.BlockSpec((C, C), lambda i: (0, 0)),
        pl.BlockSpec((C,), lambda i: (0,)),
    ],
    out_specs=pl.BlockSpec((B, C), lambda i: (i, 0)),
)


def kernel(x, edge_index, lin_w, lin_bias, fc_w, fc_b):
    col3d = edge_index[1].reshape(NW, NCHUNK, CHUNK)
    x_p = jnp.pad(x, ((0, N_P - N), (0, 0)))

    deg_part = _deg_kernel(col3d).reshape(NC, N_P)
    g = _k2(deg_part, x_p, lin_w)
    s_part = _agg_kernel(g, edge_index[0], col3d)
    out = _k4(s_part, deg_part, lin_bias, fc_w, fc_b)
    return out[:N]
